# baseline (speedup 1.0000x reference)
import jax
import jax.numpy as jnp
from jax.experimental import pallas as pl

_IN_CH = 96
_CP = 128
_HID = 192
_OUT = 85
_GN_EPS = 1e-05
_STRIDES = (8, 16, 32, 64, 128)
_SIZES = ((64, 64), (32, 32), (16, 16), (8, 8), (4, 4))


def _one_level(x, w3_ref, rows_ref, m_ref, wf_ref, fb_ref,
               cls_ref, cent_ref, reg_ref, H, W, stride):
    hw = H * W
    xcat = jnp.concatenate(
        [x[:, 0:W, :], x[:, 1:W + 1, :], x[:, 2:W + 2, :]], axis=-1)
    acc = jnp.zeros((hw, _HID), dtype=jnp.float32)
    for ky in range(3):
        xs = xcat[ky:ky + H].reshape(hw, 3 * _CP)
        acc = acc + jnp.dot(xs, w3_ref[ky],
                            preferred_element_type=jnp.float32)
    bias = rows_ref[0:1]
    gamma = rows_ref[1:2]
    beta = rows_ref[2:3]
    s1 = jnp.sum(acc, axis=0, keepdims=True)
    s2 = jnp.sum(acc * acc, axis=0, keepdims=True)
    t1 = s1 + hw * bias
    t2 = s2 + (2.0 * bias) * s1 + hw * (bias * bias)
    g1 = jnp.dot(t1, m_ref[...], preferred_element_type=jnp.float32)
    g2 = jnp.dot(t2, m_ref[...], preferred_element_type=jnp.float32)
    n = 3.0 * hw
    mean = g1 / n
    var = g2 / n - mean * mean
    scale = jax.lax.rsqrt(var + _GN_EPS) * gamma
    shift = (bias - mean) * scale + beta
    h = acc * scale + shift
    h = h * jax.nn.sigmoid(h)
    yt = jax.lax.dot_general(wf_ref[...], h, (((0,), (1,)), ((), ())),
                             preferred_element_type=jnp.float32)
    yt = yt + fb_ref[...]
    cls_ref[0] = yt[0:80]
    cent_ref[0] = yt[80:81]
    reg_ref[0] = jnp.maximum(yt[81:85] * float(stride), 0.0)


def _fused_kernel(x0, x1, x2, x3, x4, w3_ref, rows_ref, m_ref, wf_ref,
                  fb_ref, *out_refs):
    xs = (x0, x1, x2, x3, x4)
    for i, ((H, W), stride, xr) in enumerate(zip(_SIZES, _STRIDES, xs)):
        _one_level(xr[0], w3_ref, rows_ref, m_ref, wf_ref, fb_ref,
                   out_refs[3 * i], out_refs[3 * i + 1], out_refs[3 * i + 2],
                   H, W, stride)


def kernel(fpn0, fpn1, fpn2, fpn3, fpn4,
           cls_w, cls_b, cls_g, cls_beta, cls_fw, cls_fb,
           reg_w, reg_b, reg_g, reg_beta, reg_fw, reg_fb):
    fpn = (fpn0, fpn1, fpn2, fpn3, fpn4)
    B = fpn0.shape[0]

    def taps(w):
        return jnp.transpose(w, (2, 3, 1, 0))
    w3 = jnp.concatenate([taps(cls_w), taps(reg_w)], axis=-1)
    w3 = jnp.pad(w3, ((0, 0), (0, 0), (0, _CP - _IN_CH), (0, 0)))
    w3 = w3.reshape(3, 3 * _CP, _HID)
    rows = jnp.stack([
        jnp.concatenate([cls_b, reg_b]),
        jnp.concatenate([cls_g, reg_g]),
        jnp.concatenate([cls_beta, reg_beta]),
    ], axis=0)
    ids = jnp.arange(_HID) // 3
    m = (ids[:, None] == ids[None, :]).astype(jnp.float32)
    wf = jnp.zeros((_HID, _OUT), jnp.float32)
    wf = wf.at[:_IN_CH, :80].set(jnp.transpose(cls_fw.reshape(80, _IN_CH)))
    wf = wf.at[_IN_CH:, 80:].set(jnp.transpose(reg_fw.reshape(5, _IN_CH)))
    fb = jnp.concatenate([cls_fb, reg_fb])[:, None]

    xps, in_specs, out_specs, out_shapes = [], [], [], []
    for (H, W), x in zip(_SIZES, fpn):
        xp = jnp.pad(jnp.transpose(x, (0, 2, 3, 1)),
                     ((0, 0), (1, 1), (1, 1), (0, _CP - _IN_CH)))
        xps.append(xp)
        in_specs.append(
            pl.BlockSpec((1, H + 2, W + 2, _CP), lambda b: (b, 0, 0, 0)))
        hw = H * W
        for c in (80, 1, 4):
            out_specs.append(pl.BlockSpec((1, c, hw), lambda b: (b, 0, 0)))
            out_shapes.append(jax.ShapeDtypeStruct((B, c, hw), jnp.float32))
    in_specs += [
        pl.BlockSpec((3, 3 * _CP, _HID), lambda b: (0, 0, 0)),
        pl.BlockSpec((3, _HID), lambda b: (0, 0)),
        pl.BlockSpec((_HID, _HID), lambda b: (0, 0)),
        pl.BlockSpec((_HID, _OUT), lambda b: (0, 0)),
        pl.BlockSpec((_OUT, 1), lambda b: (0, 0)),
    ]

    outs = pl.pallas_call(
        _fused_kernel,
        grid=(B,),
        in_specs=in_specs,
        out_specs=out_specs,
        out_shape=out_shapes,
    )(*xps, w3, rows, m, wf, fb)

    cls_out, reg_out, cent_out = [], [], []
    for i, (H, W) in enumerate(_SIZES):
        cls_out.append(outs[3 * i].reshape(B, 80, H, W))
        cent_out.append(outs[3 * i + 1].reshape(B, 1, H, W))
        reg_out.append(outs[3 * i + 2].reshape(B, 4, H, W))
    return tuple(cls_out) + tuple(reg_out) + tuple(cent_out)


# speedup vs baseline: 1.0342x; 1.0342x over previous
import jax
import jax.numpy as jnp
from jax.experimental import pallas as pl

_IN_CH = 96
_CP = 128
_HID = 192
_OUT = 85
_GN_EPS = 1e-05
_STRIDES = (8, 16, 32, 64, 128)
_SIZES = ((64, 64), (32, 32), (16, 16), (8, 8), (4, 4))


def _one_level(x, w3_ref, rows_ref, m_ref, wf_ref, fb_ref,
               cls_ref, cent_ref, reg_ref, H, W, stride):
    hw = H * W
    xcat = jnp.concatenate(
        [x[:, 0:W, :], x[:, 1:W + 1, :], x[:, 2:W + 2, :]], axis=-1)
    acc = jnp.zeros((hw, _HID), dtype=jnp.float32)
    for ky in range(3):
        xs = xcat[ky:ky + H].reshape(hw, 3 * _CP)
        acc = acc + jnp.dot(xs, w3_ref[ky],
                            preferred_element_type=jnp.float32)
    bias = rows_ref[0:1]
    gamma = rows_ref[1:2]
    beta = rows_ref[2:3]
    s1 = jnp.sum(acc, axis=0, keepdims=True)
    s2 = jnp.sum(acc * acc, axis=0, keepdims=True)
    t1 = s1 + hw * bias
    t2 = s2 + (2.0 * bias) * s1 + hw * (bias * bias)
    g1 = jnp.dot(t1, m_ref[...], preferred_element_type=jnp.float32)
    g2 = jnp.dot(t2, m_ref[...], preferred_element_type=jnp.float32)
    n = 3.0 * hw
    mean = g1 / n
    var = g2 / n - mean * mean
    scale = jax.lax.rsqrt(var + _GN_EPS) * gamma
    shift = (bias - mean) * scale + beta
    h = acc * scale + shift
    h = h * jax.nn.sigmoid(h)
    yt = jax.lax.dot_general(wf_ref[...], h, (((0,), (1,)), ((), ())),
                             preferred_element_type=jnp.float32)
    yt = yt + fb_ref[...]
    cls_ref[0] = yt[0:80]
    cent_ref[0] = yt[80:81]
    reg_ref[0] = jnp.maximum(yt[81:85] * float(stride), 0.0)


def _fused_kernel(x0, x1, x2, x3, x4, w3_ref, rows_ref, m_ref, wf_ref,
                  fb_ref, *out_refs):
    xs = (x0, x1, x2, x3, x4)
    for i, ((H, W), stride, xr) in enumerate(zip(_SIZES, _STRIDES, xs)):
        _one_level(xr[0], w3_ref, rows_ref, m_ref, wf_ref, fb_ref,
                   out_refs[3 * i], out_refs[3 * i + 1], out_refs[3 * i + 2],
                   H, W, stride)


def kernel(fpn0, fpn1, fpn2, fpn3, fpn4,
           cls_w, cls_b, cls_g, cls_beta, cls_fw, cls_fb,
           reg_w, reg_b, reg_g, reg_beta, reg_fw, reg_fb):
    fpn = (fpn0, fpn1, fpn2, fpn3, fpn4)
    B = fpn0.shape[0]

    def taps(w):
        return jnp.transpose(w, (2, 3, 1, 0))
    w3 = jnp.concatenate([taps(cls_w), taps(reg_w)], axis=-1)
    w3 = jnp.pad(w3, ((0, 0), (0, 0), (0, _CP - _IN_CH), (0, 0)))
    w3 = w3.reshape(3, 3 * _CP, _HID).astype(jnp.bfloat16)
    rows = jnp.stack([
        jnp.concatenate([cls_b, reg_b]),
        jnp.concatenate([cls_g, reg_g]),
        jnp.concatenate([cls_beta, reg_beta]),
    ], axis=0)
    ids = jnp.arange(_HID) // 3
    m = (ids[:, None] == ids[None, :]).astype(jnp.float32)
    wf = jnp.zeros((_HID, _OUT), jnp.float32)
    wf = wf.at[:_IN_CH, :80].set(jnp.transpose(cls_fw.reshape(80, _IN_CH)))
    wf = wf.at[_IN_CH:, 80:].set(jnp.transpose(reg_fw.reshape(5, _IN_CH)))
    fb = jnp.concatenate([cls_fb, reg_fb])[:, None]

    xps, in_specs, out_specs, out_shapes = [], [], [], []
    for (H, W), x in zip(_SIZES, fpn):
        xp = jnp.pad(jnp.transpose(x, (0, 2, 3, 1)),
                     ((0, 0), (1, 1), (1, 1), (0, _CP - _IN_CH)))
        xp = xp.astype(jnp.bfloat16)
        xps.append(xp)
        in_specs.append(
            pl.BlockSpec((1, H + 2, W + 2, _CP), lambda b: (b, 0, 0, 0)))
        hw = H * W
        for c in (80, 1, 4):
            out_specs.append(pl.BlockSpec((1, c, hw), lambda b: (b, 0, 0)))
            out_shapes.append(jax.ShapeDtypeStruct((B, c, hw), jnp.float32))
    in_specs += [
        pl.BlockSpec((3, 3 * _CP, _HID), lambda b: (0, 0, 0)),
        pl.BlockSpec((3, _HID), lambda b: (0, 0)),
        pl.BlockSpec((_HID, _HID), lambda b: (0, 0)),
        pl.BlockSpec((_HID, _OUT), lambda b: (0, 0)),
        pl.BlockSpec((_OUT, 1), lambda b: (0, 0)),
    ]

    outs = pl.pallas_call(
        _fused_kernel,
        grid=(B,),
        in_specs=in_specs,
        out_specs=out_specs,
        out_shape=out_shapes,
    )(*xps, w3, rows, m, wf, fb)

    cls_out, reg_out, cent_out = [], [], []
    for i, (H, W) in enumerate(_SIZES):
        cls_out.append(outs[3 * i].reshape(B, 80, H, W))
        cent_out.append(outs[3 * i + 1].reshape(B, 1, H, W))
        reg_out.append(outs[3 * i + 2].reshape(B, 4, H, W))
    return tuple(cls_out) + tuple(reg_out) + tuple(cent_out)


# speedup vs baseline: 1.0347x; 1.0005x over previous
import jax
import jax.numpy as jnp
from jax.experimental import pallas as pl

_IN_CH = 96
_CP = 128
_HID = 192
_OUT = 85
_GN_EPS = 1e-05
_STRIDES = (8, 16, 32, 64, 128)
_SIZES = ((64, 64), (32, 32), (16, 16), (8, 8), (4, 4))


def _one_level(x, w3_ref, rows_ref, m_ref, wf_ref, fb_ref,
               cls_ref, cent_ref, reg_ref, H, W, stride):
    hw = H * W
    xcat = jnp.concatenate(
        [x[:, 0:W, :], x[:, 1:W + 1, :], x[:, 2:W + 2, :]], axis=-1)
    acc = jnp.zeros((hw, _HID), dtype=jnp.float32)
    for ky in range(3):
        xs = xcat[ky:ky + H].reshape(hw, 3 * _CP)
        acc = acc + jnp.dot(xs, w3_ref[ky],
                            preferred_element_type=jnp.float32)
    bias = rows_ref[0:1]
    gamma = rows_ref[1:2]
    beta = rows_ref[2:3]
    s1 = jnp.sum(acc, axis=0, keepdims=True)
    s2 = jnp.sum(acc * acc, axis=0, keepdims=True)
    t1 = s1 + hw * bias
    t2 = s2 + (2.0 * bias) * s1 + hw * (bias * bias)
    g1 = jnp.dot(t1, m_ref[...], preferred_element_type=jnp.float32)
    g2 = jnp.dot(t2, m_ref[...], preferred_element_type=jnp.float32)
    n = 3.0 * hw
    mean = g1 / n
    var = g2 / n - mean * mean
    scale = jax.lax.rsqrt(var + _GN_EPS) * gamma
    shift = (bias - mean) * scale + beta
    h = acc * scale + shift
    h = (h * jax.nn.sigmoid(h)).astype(jnp.bfloat16)
    yt = jax.lax.dot_general(wf_ref[...], h, (((0,), (1,)), ((), ())),
                             preferred_element_type=jnp.float32)
    yt = yt + fb_ref[...]
    cls_ref[0] = yt[0:80]
    cent_ref[0] = yt[80:81]
    reg_ref[0] = jnp.maximum(yt[81:85] * float(stride), 0.0)


def _fused_kernel(x0, x1, x2, x3, x4, w3_ref, rows_ref, m_ref, wf_ref,
                  fb_ref, *out_refs):
    xs = (x0, x1, x2, x3, x4)
    for i, ((H, W), stride, xr) in enumerate(zip(_SIZES, _STRIDES, xs)):
        _one_level(xr[0], w3_ref, rows_ref, m_ref, wf_ref, fb_ref,
                   out_refs[3 * i], out_refs[3 * i + 1], out_refs[3 * i + 2],
                   H, W, stride)


def kernel(fpn0, fpn1, fpn2, fpn3, fpn4,
           cls_w, cls_b, cls_g, cls_beta, cls_fw, cls_fb,
           reg_w, reg_b, reg_g, reg_beta, reg_fw, reg_fb):
    fpn = (fpn0, fpn1, fpn2, fpn3, fpn4)
    B = fpn0.shape[0]

    def taps(w):
        return jnp.transpose(w, (2, 3, 1, 0))
    w3 = jnp.concatenate([taps(cls_w), taps(reg_w)], axis=-1)
    w3 = jnp.pad(w3, ((0, 0), (0, 0), (0, _CP - _IN_CH), (0, 0)))
    w3 = w3.reshape(3, 3 * _CP, _HID).astype(jnp.bfloat16)
    rows = jnp.stack([
        jnp.concatenate([cls_b, reg_b]),
        jnp.concatenate([cls_g, reg_g]),
        jnp.concatenate([cls_beta, reg_beta]),
    ], axis=0)
    ids = jnp.arange(_HID) // 3
    m = (ids[:, None] == ids[None, :]).astype(jnp.float32)
    wf = jnp.zeros((_HID, _OUT), jnp.float32)
    wf = wf.at[:_IN_CH, :80].set(jnp.transpose(cls_fw.reshape(80, _IN_CH)))
    wf = wf.at[_IN_CH:, 80:].set(jnp.transpose(reg_fw.reshape(5, _IN_CH)))
    wf = wf.astype(jnp.bfloat16)
    fb = jnp.concatenate([cls_fb, reg_fb])[:, None]

    xps, in_specs, out_specs, out_shapes = [], [], [], []
    for (H, W), x in zip(_SIZES, fpn):
        xp = jnp.pad(jnp.transpose(x, (0, 2, 3, 1)),
                     ((0, 0), (1, 1), (1, 1), (0, _CP - _IN_CH)))
        xp = xp.astype(jnp.bfloat16)
        xps.append(xp)
        in_specs.append(
            pl.BlockSpec((1, H + 2, W + 2, _CP), lambda b: (b, 0, 0, 0)))
        hw = H * W
        for c in (80, 1, 4):
            out_specs.append(pl.BlockSpec((1, c, hw), lambda b: (b, 0, 0)))
            out_shapes.append(jax.ShapeDtypeStruct((B, c, hw), jnp.float32))
    in_specs += [
        pl.BlockSpec((3, 3 * _CP, _HID), lambda b: (0, 0, 0)),
        pl.BlockSpec((3, _HID), lambda b: (0, 0)),
        pl.BlockSpec((_HID, _HID), lambda b: (0, 0)),
        pl.BlockSpec((_HID, _OUT), lambda b: (0, 0)),
        pl.BlockSpec((_OUT, 1), lambda b: (0, 0)),
    ]

    outs = pl.pallas_call(
        _fused_kernel,
        grid=(B,),
        in_specs=in_specs,
        out_specs=out_specs,
        out_shape=out_shapes,
    )(*xps, w3, rows, m, wf, fb)

    cls_out, reg_out, cent_out = [], [], []
    for i, (H, W) in enumerate(_SIZES):
        cls_out.append(outs[3 * i].reshape(B, 80, H, W))
        cent_out.append(outs[3 * i + 1].reshape(B, 1, H, W))
        reg_out.append(outs[3 * i + 2].reshape(B, 4, H, W))
    return tuple(cls_out) + tuple(reg_out) + tuple(cent_out)


# speedup vs baseline: 1.0380x; 1.0031x over previous
import jax
import jax.numpy as jnp
from jax.experimental import pallas as pl
from jax.experimental.pallas import tpu as pltpu

_IN_CH = 96
_CP = 128
_HID = 192
_OUT = 85
_GN_EPS = 1e-05
_STRIDES = (8, 16, 32, 64, 128)
_SIZES = ((64, 64), (32, 32), (16, 16), (8, 8), (4, 4))


def _one_level(x, w3_ref, rows_ref, m_ref, wf_ref, fb_ref,
               cls_ref, cent_ref, reg_ref, H, W, stride):
    hw = H * W
    xcat = jnp.concatenate(
        [x[:, 0:W, :], x[:, 1:W + 1, :], x[:, 2:W + 2, :]], axis=-1)
    acc = jnp.zeros((hw, _HID), dtype=jnp.float32)
    for ky in range(3):
        xs = xcat[ky:ky + H].reshape(hw, 3 * _CP)
        acc = acc + jnp.dot(xs, w3_ref[ky],
                            preferred_element_type=jnp.float32)
    bias = rows_ref[0:1]
    gamma = rows_ref[1:2]
    beta = rows_ref[2:3]
    s1 = jnp.sum(acc, axis=0, keepdims=True)
    s2 = jnp.sum(acc * acc, axis=0, keepdims=True)
    t1 = s1 + hw * bias
    t2 = s2 + (2.0 * bias) * s1 + hw * (bias * bias)
    g1 = jnp.dot(t1, m_ref[...], preferred_element_type=jnp.float32)
    g2 = jnp.dot(t2, m_ref[...], preferred_element_type=jnp.float32)
    n = 3.0 * hw
    mean = g1 / n
    var = g2 / n - mean * mean
    scale = jax.lax.rsqrt(var + _GN_EPS) * gamma
    shift = (bias - mean) * scale + beta
    h = acc * scale + shift
    h = (h * jax.nn.sigmoid(h)).astype(jnp.bfloat16)
    yt = jax.lax.dot_general(wf_ref[...], h, (((0,), (1,)), ((), ())),
                             preferred_element_type=jnp.float32)
    yt = yt + fb_ref[...]
    cls_ref[0] = yt[0:80]
    cent_ref[0] = yt[80:81]
    reg_ref[0] = jnp.maximum(yt[81:85] * float(stride), 0.0)


def _fused_kernel(x0, x1, x2, x3, x4, w3_ref, rows_ref, m_ref, wf_ref,
                  fb_ref, *out_refs):
    xs = (x0, x1, x2, x3, x4)
    for i, ((H, W), stride, xr) in enumerate(zip(_SIZES, _STRIDES, xs)):
        _one_level(xr[0], w3_ref, rows_ref, m_ref, wf_ref, fb_ref,
                   out_refs[3 * i], out_refs[3 * i + 1], out_refs[3 * i + 2],
                   H, W, stride)


def kernel(fpn0, fpn1, fpn2, fpn3, fpn4,
           cls_w, cls_b, cls_g, cls_beta, cls_fw, cls_fb,
           reg_w, reg_b, reg_g, reg_beta, reg_fw, reg_fb):
    fpn = (fpn0, fpn1, fpn2, fpn3, fpn4)
    B = fpn0.shape[0]

    def taps(w):
        return jnp.transpose(w, (2, 3, 1, 0))
    w3 = jnp.concatenate([taps(cls_w), taps(reg_w)], axis=-1)
    w3 = jnp.pad(w3, ((0, 0), (0, 0), (0, _CP - _IN_CH), (0, 0)))
    w3 = w3.reshape(3, 3 * _CP, _HID).astype(jnp.bfloat16)
    rows = jnp.stack([
        jnp.concatenate([cls_b, reg_b]),
        jnp.concatenate([cls_g, reg_g]),
        jnp.concatenate([cls_beta, reg_beta]),
    ], axis=0)
    ids = jnp.arange(_HID) // 3
    m = (ids[:, None] == ids[None, :]).astype(jnp.float32)
    wf = jnp.zeros((_HID, _OUT), jnp.float32)
    wf = wf.at[:_IN_CH, :80].set(jnp.transpose(cls_fw.reshape(80, _IN_CH)))
    wf = wf.at[_IN_CH:, 80:].set(jnp.transpose(reg_fw.reshape(5, _IN_CH)))
    wf = wf.astype(jnp.bfloat16)
    fb = jnp.concatenate([cls_fb, reg_fb])[:, None]

    xps, in_specs, out_specs, out_shapes = [], [], [], []
    for (H, W), x in zip(_SIZES, fpn):
        xp = jnp.pad(jnp.transpose(x, (0, 2, 3, 1)),
                     ((0, 0), (1, 1), (1, 1), (0, _CP - _IN_CH)))
        xp = xp.astype(jnp.bfloat16)
        xps.append(xp)
        in_specs.append(
            pl.BlockSpec((1, H + 2, W + 2, _CP), lambda b: (b, 0, 0, 0)))
        hw = H * W
        for c in (80, 1, 4):
            out_specs.append(pl.BlockSpec((1, c, hw), lambda b: (b, 0, 0)))
            out_shapes.append(jax.ShapeDtypeStruct((B, c, hw), jnp.float32))
    in_specs += [
        pl.BlockSpec((3, 3 * _CP, _HID), lambda b: (0, 0, 0)),
        pl.BlockSpec((3, _HID), lambda b: (0, 0)),
        pl.BlockSpec((_HID, _HID), lambda b: (0, 0)),
        pl.BlockSpec((_HID, _OUT), lambda b: (0, 0)),
        pl.BlockSpec((_OUT, 1), lambda b: (0, 0)),
    ]

    outs = pl.pallas_call(
        _fused_kernel,
        grid=(B,),
        in_specs=in_specs,
        out_specs=out_specs,
        out_shape=out_shapes,
        compiler_params=pltpu.CompilerParams(
            dimension_semantics=("parallel",)),
    )(*xps, w3, rows, m, wf, fb)

    cls_out, reg_out, cent_out = [], [], []
    for i, (H, W) in enumerate(_SIZES):
        cls_out.append(outs[3 * i].reshape(B, 80, H, W))
        cent_out.append(outs[3 * i + 1].reshape(B, 1, H, W))
        reg_out.append(outs[3 * i + 2].reshape(B, 4, H, W))
    return tuple(cls_out) + tuple(reg_out) + tuple(cent_out)
